# baseline (speedup 1.0000x reference)
import functools

import jax
import jax.numpy as jnp
from jax import lax
from jax.experimental import pallas as pl
from jax.experimental.pallas import tpu as pltpu
from jax.experimental.pallas import tpu_sc as plsc

_DIM = 650000
_OBS = 64

_NC = 2
_NS = 16
_CHUNK = 40704
_NPAD = _NS * _CHUNK


def _scatter_labels(idx, vals):
    mesh = plsc.VectorSubcoreMesh(core_axis_name="c", subcore_axis_name="s")

    @functools.partial(
        pl.kernel,
        mesh=mesh,
        out_type=jax.ShapeDtypeStruct((_NPAD,), jnp.int32),
        scratch_types=[
            pltpu.VMEM((_CHUNK,), jnp.int32),
            pltpu.VMEM((_CHUNK,), jnp.int32),
            pltpu.VMEM_SHARED((_NPAD,), jnp.int32),
            pltpu.SemaphoreType.DMA,
        ],
    )
    def sc_kernel(idx_hbm, vals_hbm, out_hbm, idx_v, vals_v, lab_sh, sem):
        c = lax.axis_index("c")
        s = lax.axis_index("s")

        @pl.when(c == 0)
        def _():
            pltpu.sync_copy(idx_hbm.at[s], idx_v)
            pltpu.sync_copy(vals_hbm.at[s], vals_v)
            pltpu.async_copy(vals_v, lab_sh.at[idx_v], sem).wait()
            plsc.subcore_barrier()
            sl = pl.ds(s * _CHUNK, _CHUNK)
            pltpu.sync_copy(lab_sh.at[sl], out_hbm.at[sl])

    return sc_kernel(idx, vals)


def _apply_links(x, labels):
    bc = 16384
    grid = pl.cdiv(_DIM, bc)

    def body(lab_ref, x_ref, o_ref):
        lab = lab_ref[...]
        xx = x_ref[...]
        e = jnp.exp(xx)
        sig = 1.0 - 1.0 / (1.0 + e)
        o_ref[...] = jnp.where(lab == 1, sig, jnp.where(lab == 2, e, xx))

    return pl.pallas_call(
        body,
        grid=(grid,),
        in_specs=[
            pl.BlockSpec((1, bc), lambda i: (0, i)),
            pl.BlockSpec((_OBS, bc), lambda i: (0, i)),
        ],
        out_specs=pl.BlockSpec((_OBS, bc), lambda i: (0, i)),
        out_shape=jax.ShapeDtypeStruct((_OBS, _DIM), jnp.float32),
        compiler_params=pltpu.CompilerParams(
            dimension_semantics=("parallel",)
        ),
    )(labels.reshape(1, _NPAD), x)


def kernel(x, id_gauss, id_bern, id_pois):
    idx = jnp.concatenate(
        [
            id_gauss.astype(jnp.int32),
            id_bern.astype(jnp.int32),
            id_pois.astype(jnp.int32),
        ]
    )
    vals = jnp.concatenate(
        [
            jnp.zeros(id_gauss.shape[0], jnp.int32),
            jnp.ones(id_bern.shape[0], jnp.int32),
            jnp.full(id_pois.shape[0], 2, jnp.int32),
        ]
    )
    pad = _NPAD - _DIM
    idx = jnp.concatenate([idx, jnp.broadcast_to(idx[-1], (pad,))])
    vals = jnp.concatenate([vals, jnp.full((pad,), 2, jnp.int32)])
    labels = _scatter_labels(
        idx.reshape(_NS, _CHUNK), vals.reshape(_NS, _CHUNK)
    )
    return _apply_links(x, labels)


# speedup vs baseline: 1.0255x; 1.0255x over previous
import functools

import jax
import jax.numpy as jnp
from jax import lax
from jax.experimental import pallas as pl
from jax.experimental.pallas import tpu as pltpu
from jax.experimental.pallas import tpu_sc as plsc

_DIM = 650000
_OBS = 64

_NC = 2
_NS = 16
_CHUNK = 40704
_NPAD = _NS * _CHUNK


def _scatter_labels(idx, vals):
    mesh = plsc.VectorSubcoreMesh(core_axis_name="c", subcore_axis_name="s")

    @functools.partial(
        pl.kernel,
        mesh=mesh,
        out_type=jax.ShapeDtypeStruct((_NPAD,), jnp.int32),
        scratch_types=[
            pltpu.VMEM((_CHUNK,), jnp.int32),
            pltpu.VMEM((_CHUNK,), jnp.int32),
            pltpu.VMEM_SHARED((_NPAD,), jnp.int32),
            pltpu.SemaphoreType.DMA,
        ],
    )
    def sc_kernel(idx_hbm, vals_hbm, out_hbm, idx_v, vals_v, lab_sh, sem):
        c = lax.axis_index("c")
        s = lax.axis_index("s")

        @pl.when(c == 0)
        def _():
            pltpu.sync_copy(idx_hbm.at[s], idx_v)
            pltpu.sync_copy(vals_hbm.at[s], vals_v)
            pltpu.async_copy(vals_v, lab_sh.at[idx_v], sem).wait()
            plsc.subcore_barrier()
            sl = pl.ds(s * _CHUNK, _CHUNK)
            pltpu.sync_copy(lab_sh.at[sl], out_hbm.at[sl])

    return sc_kernel(idx, vals)


def _apply_links(x, labels):
    bc = 49152
    grid = pl.cdiv(_DIM, bc)

    def body(lab_ref, x_ref, o_ref):
        lab = lab_ref[...]
        xx = x_ref[...]
        e = jnp.exp(xx)
        sig = 1.0 - 1.0 / (1.0 + e)
        o_ref[...] = jnp.where(lab == 1, sig, jnp.where(lab == 2, e, xx))

    return pl.pallas_call(
        body,
        grid=(grid,),
        in_specs=[
            pl.BlockSpec((1, bc), lambda i: (0, i)),
            pl.BlockSpec((_OBS, bc), lambda i: (0, i)),
        ],
        out_specs=pl.BlockSpec((_OBS, bc), lambda i: (0, i)),
        out_shape=jax.ShapeDtypeStruct((_OBS, _DIM), jnp.float32),
        compiler_params=pltpu.CompilerParams(
            dimension_semantics=("parallel",)
        ),
    )(labels.reshape(1, _NPAD), x)


def kernel(x, id_gauss, id_bern, id_pois):
    idx = jnp.concatenate(
        [
            id_gauss.astype(jnp.int32),
            id_bern.astype(jnp.int32),
            id_pois.astype(jnp.int32),
        ]
    )
    vals = jnp.concatenate(
        [
            jnp.zeros(id_gauss.shape[0], jnp.int32),
            jnp.ones(id_bern.shape[0], jnp.int32),
            jnp.full(id_pois.shape[0], 2, jnp.int32),
        ]
    )
    pad = _NPAD - _DIM
    idx = jnp.concatenate([idx, jnp.broadcast_to(idx[-1], (pad,))])
    vals = jnp.concatenate([vals, jnp.full((pad,), 2, jnp.int32)])
    labels = _scatter_labels(
        idx.reshape(_NS, _CHUNK), vals.reshape(_NS, _CHUNK)
    )
    return _apply_links(x, labels)
